# baseline (device time: 26537 ns/iter reference)
import jax
import jax.numpy as jnp
from jax import lax
from jax.experimental import pallas as pl
from jax.experimental.pallas import tpu as pltpu

N_DEV = 4
STAT_SUB = 16
STAT_LANE = 128
CI = 8
CO = 8


def kernel(x):
    m_rows, n_cols = x.shape
    assert m_rows == STAT_SUB * STAT_LANE
    rows_i = m_rows // CI
    rows_o = m_rows // CO

    def body(
        x_hbm, out_hbm, e_ref, s_ref, gather_ref,
        xbuf, obuf, in_sems, out_sems, send_sems, recv_sems,
    ):
        my_pos = lax.axis_index("i")

        barrier_sem = pltpu.get_barrier_semaphore()
        for off in range(1, N_DEV):
            peer = (my_pos + off) % N_DEV
            pl.semaphore_signal(
                barrier_sem, inc=1,
                device_id=(peer,), device_id_type=pl.DeviceIdType.MESH,
            )

        def in_copy(c):
            return pltpu.make_async_copy(
                x_hbm.at[pl.ds(c * rows_i, rows_i), :],
                xbuf.at[c % 2],
                in_sems.at[c % 2],
            )

        in_copy(0).start()
        for c in range(CI):
            if c + 1 < CI:
                in_copy(c + 1).start()
            in_copy(c).wait()
            sl = pl.ds(c * rows_i, rows_i)
            ev = jnp.exp(xbuf[c % 2])
            e_ref[sl, :] = ev.astype(jnp.bfloat16)
            s_ref[sl, :] = jnp.sum(ev, axis=1, keepdims=True)

        gather_ref[my_pos] = jnp.swapaxes(s_ref[...], 0, 1).reshape(
            STAT_SUB, STAT_LANE
        )

        pl.semaphore_wait(barrier_sem, N_DEV - 1)

        sends = []
        for off in range(1, N_DEV):
            peer = (my_pos + off) % N_DEV
            rdma = pltpu.make_async_remote_copy(
                src_ref=gather_ref.at[my_pos],
                dst_ref=gather_ref.at[my_pos],
                send_sem=send_sems.at[off],
                recv_sem=recv_sems.at[my_pos],
                device_id=(peer,),
                device_id_type=pl.DeviceIdType.MESH,
            )
            rdma.start()
            sends.append(rdma)

        for off in range(1, N_DEV):
            src = (my_pos + off) % N_DEV
            recv = pltpu.make_async_remote_copy(
                src_ref=gather_ref.at[src],
                dst_ref=gather_ref.at[src],
                send_sem=send_sems.at[0],
                recv_sem=recv_sems.at[src],
                device_id=(src,),
                device_id_type=pl.DeviceIdType.MESH,
            )
            recv.wait_recv()
        for rdma in sends:
            rdma.wait_send()

        s16 = (gather_ref[0] + gather_ref[1]
               + gather_ref[2] + gather_ref[3])
        inv_col = jnp.swapaxes((1.0 / s16).reshape(1, m_rows), 0, 1)

        def out_copy(c):
            return pltpu.make_async_copy(
                obuf.at[c % 2],
                out_hbm.at[pl.ds(c * rows_o, rows_o), :],
                out_sems.at[c % 2],
            )

        out_copies = []
        for c in range(CO):
            if c >= 2:
                out_copies[c - 2].wait()
            sl = pl.ds(c * rows_o, rows_o)
            obuf[c % 2] = (
                e_ref[sl, :].astype(jnp.float32)
                * inv_col[c * rows_o:(c + 1) * rows_o, :]
            ).astype(jnp.bfloat16)
            cp = out_copy(c)
            cp.start()
            out_copies.append(cp)
        out_copies[CO - 2].wait()
        out_copies[CO - 1].wait()

    return pl.pallas_call(
        body,
        out_shape=jax.ShapeDtypeStruct((m_rows, n_cols), jnp.bfloat16),
        in_specs=[pl.BlockSpec(memory_space=pl.ANY)],
        out_specs=pl.BlockSpec(memory_space=pl.ANY),
        scratch_shapes=[
            pltpu.VMEM((m_rows, n_cols), jnp.bfloat16),
            pltpu.VMEM((m_rows, 1), jnp.float32),
            pltpu.VMEM((N_DEV, STAT_SUB, STAT_LANE), jnp.float32),
            pltpu.VMEM((2, m_rows // CI, n_cols), jnp.float32),
            pltpu.VMEM((2, m_rows // CO, n_cols), jnp.bfloat16),
            pltpu.SemaphoreType.DMA((2,)),
            pltpu.SemaphoreType.DMA((2,)),
            pltpu.SemaphoreType.DMA((N_DEV,)),
            pltpu.SemaphoreType.DMA((N_DEV,)),
        ],
        compiler_params=pltpu.CompilerParams(
            collective_id=0, vmem_limit_bytes=64 * 1024 * 1024
        ),
    )(x)


# device time: 24576 ns/iter; 1.0798x vs baseline; 1.0798x over previous
import jax
import jax.numpy as jnp
from jax import lax
from jax.experimental import pallas as pl
from jax.experimental.pallas import tpu as pltpu

N_DEV = 4
H = 2
STAT_LANE = 128


def kernel(x):
    m_rows, n_cols = x.shape
    rows_h = m_rows // H
    stat_sub = rows_h // STAT_LANE

    def body(x_ref, out_ref, e_ref, gather_ref, send_sems, recv_sems):
        my_pos = lax.axis_index("i")

        barrier_sem = pltpu.get_barrier_semaphore()
        for off in range(1, N_DEV):
            peer = (my_pos + off) % N_DEV
            pl.semaphore_signal(
                barrier_sem, inc=1,
                device_id=(peer,), device_id_type=pl.DeviceIdType.MESH,
            )

        def send_half(h):
            res = []
            for off in range(1, N_DEV):
                peer = (my_pos + off) % N_DEV
                rdma = pltpu.make_async_remote_copy(
                    src_ref=gather_ref.at[h, my_pos],
                    dst_ref=gather_ref.at[h, my_pos],
                    send_sem=send_sems.at[h * N_DEV + off],
                    recv_sem=recv_sems.at[h * N_DEV + my_pos],
                    device_id=(peer,),
                    device_id_type=pl.DeviceIdType.MESH,
                )
                rdma.start()
                res.append(rdma)
            return res

        def wait_half(h):
            for off in range(1, N_DEV):
                src = (my_pos + off) % N_DEV
                recv = pltpu.make_async_remote_copy(
                    src_ref=gather_ref.at[h, src],
                    dst_ref=gather_ref.at[h, src],
                    send_sem=send_sems.at[h * N_DEV],
                    recv_sem=recv_sems.at[h * N_DEV + src],
                    device_id=(src,),
                    device_id_type=pl.DeviceIdType.MESH,
                )
                recv.wait_recv()

        sends = []
        for h in range(H):
            sl = pl.ds(h * rows_h, rows_h)
            ev = jnp.exp(x_ref[sl, :])
            s_col = jnp.sum(ev, axis=1, keepdims=True)
            e_ref[sl, :] = ev.astype(jnp.bfloat16)
            gather_ref[h, my_pos] = jnp.swapaxes(s_col, 0, 1).reshape(
                stat_sub, STAT_LANE
            )
            if h == 0:
                pl.semaphore_wait(barrier_sem, N_DEV - 1)
            sends += send_half(h)

        for h in range(H):
            wait_half(h)
            s16 = (gather_ref[h, 0] + gather_ref[h, 1]
                   + gather_ref[h, 2] + gather_ref[h, 3])
            inv_col = jnp.swapaxes((1.0 / s16).reshape(1, rows_h), 0, 1)
            sl = pl.ds(h * rows_h, rows_h)
            out_ref[sl, :] = (
                e_ref[sl, :].astype(jnp.float32) * inv_col
            ).astype(jnp.bfloat16)

        for rdma in sends:
            rdma.wait_send()

    return pl.pallas_call(
        body,
        out_shape=jax.ShapeDtypeStruct((m_rows, n_cols), jnp.bfloat16),
        in_specs=[pl.BlockSpec(memory_space=pltpu.VMEM)],
        out_specs=pl.BlockSpec(memory_space=pltpu.VMEM),
        scratch_shapes=[
            pltpu.VMEM((m_rows, n_cols), jnp.bfloat16),
            pltpu.VMEM((H, N_DEV, m_rows // H // STAT_LANE, STAT_LANE),
                       jnp.float32),
            pltpu.SemaphoreType.DMA((H * N_DEV,)),
            pltpu.SemaphoreType.DMA((H * N_DEV,)),
        ],
        compiler_params=pltpu.CompilerParams(
            collective_id=0, vmem_limit_bytes=64 * 1024 * 1024
        ),
    )(x)


# device time: 24568 ns/iter; 1.0801x vs baseline; 1.0003x over previous
import jax
import jax.numpy as jnp
from jax import lax
from jax.experimental import pallas as pl
from jax.experimental.pallas import tpu as pltpu

N_DEV = 4
H = 2
STAT_LANE = 128


def kernel(x):
    m_rows, n_cols = x.shape
    rows_h = m_rows // H
    stat_sub = rows_h // STAT_LANE

    def body(x_ref, out_ref, e_ref, gather_ref, send_sems, recv_sems):
        my_pos = lax.axis_index("i")

        barrier_sem = pltpu.get_barrier_semaphore()
        for off in range(1, N_DEV):
            peer = (my_pos + off) % N_DEV
            pl.semaphore_signal(
                barrier_sem, inc=1,
                device_id=(peer,), device_id_type=pl.DeviceIdType.MESH,
            )

        def send_half(h):
            res = []
            for off in range(1, N_DEV):
                peer = (my_pos + off) % N_DEV
                rdma = pltpu.make_async_remote_copy(
                    src_ref=gather_ref.at[h, my_pos],
                    dst_ref=gather_ref.at[h, my_pos],
                    send_sem=send_sems.at[h * N_DEV + off],
                    recv_sem=recv_sems.at[h * N_DEV + my_pos],
                    device_id=(peer,),
                    device_id_type=pl.DeviceIdType.MESH,
                )
                rdma.start()
                res.append(rdma)
            return res

        def wait_half(h):
            for off in range(1, N_DEV):
                src = (my_pos + off) % N_DEV
                recv = pltpu.make_async_remote_copy(
                    src_ref=gather_ref.at[h, src],
                    dst_ref=gather_ref.at[h, src],
                    send_sem=send_sems.at[h * N_DEV],
                    recv_sem=recv_sems.at[h * N_DEV + src],
                    device_id=(src,),
                    device_id_type=pl.DeviceIdType.MESH,
                )
                recv.wait_recv()

        sends = []
        for h in range(H):
            sl = pl.ds(h * rows_h, rows_h)
            ev = jnp.exp(x_ref[sl, :].astype(jnp.bfloat16))
            s_col = jnp.sum(
                ev.astype(jnp.float32), axis=1, keepdims=True
            )
            e_ref[sl, :] = ev
            gather_ref[h, my_pos] = jnp.swapaxes(s_col, 0, 1).reshape(
                stat_sub, STAT_LANE
            )
            if h == 0:
                pl.semaphore_wait(barrier_sem, N_DEV - 1)
            sends += send_half(h)

        for h in range(H):
            wait_half(h)
            s16 = (gather_ref[h, 0] + gather_ref[h, 1]
                   + gather_ref[h, 2] + gather_ref[h, 3])
            inv_col = jnp.swapaxes(
                (1.0 / s16).reshape(1, rows_h), 0, 1
            ).astype(jnp.bfloat16)
            sl = pl.ds(h * rows_h, rows_h)
            out_ref[sl, :] = e_ref[sl, :] * inv_col

        for rdma in sends:
            rdma.wait_send()

    return pl.pallas_call(
        body,
        out_shape=jax.ShapeDtypeStruct((m_rows, n_cols), jnp.bfloat16),
        in_specs=[pl.BlockSpec(memory_space=pltpu.VMEM)],
        out_specs=pl.BlockSpec(memory_space=pltpu.VMEM),
        scratch_shapes=[
            pltpu.VMEM((m_rows, n_cols), jnp.bfloat16),
            pltpu.VMEM((H, N_DEV, m_rows // H // STAT_LANE, STAT_LANE),
                       jnp.float32),
            pltpu.SemaphoreType.DMA((H * N_DEV,)),
            pltpu.SemaphoreType.DMA((H * N_DEV,)),
        ],
        compiler_params=pltpu.CompilerParams(
            collective_id=0, vmem_limit_bytes=64 * 1024 * 1024
        ),
    )(x)


# device time: 24501 ns/iter; 1.0831x vs baseline; 1.0027x over previous
import jax
import jax.numpy as jnp
from jax import lax
from jax.experimental import pallas as pl
from jax.experimental.pallas import tpu as pltpu

N_DEV = 4
H = 4
STAT_LANE = 128


def kernel(x):
    m_rows, n_cols = x.shape
    rows_h = m_rows // H
    stat_sub = rows_h // STAT_LANE

    def body(x_ref, out_ref, e_ref, gather_ref, send_sems, recv_sems):
        my_pos = lax.axis_index("i")

        barrier_sem = pltpu.get_barrier_semaphore()
        for off in range(1, N_DEV):
            peer = (my_pos + off) % N_DEV
            pl.semaphore_signal(
                barrier_sem, inc=1,
                device_id=(peer,), device_id_type=pl.DeviceIdType.MESH,
            )

        def send_half(h):
            res = []
            for off in range(1, N_DEV):
                peer = (my_pos + off) % N_DEV
                rdma = pltpu.make_async_remote_copy(
                    src_ref=gather_ref.at[h, my_pos],
                    dst_ref=gather_ref.at[h, my_pos],
                    send_sem=send_sems.at[h * N_DEV + off],
                    recv_sem=recv_sems.at[h * N_DEV + my_pos],
                    device_id=(peer,),
                    device_id_type=pl.DeviceIdType.MESH,
                )
                rdma.start()
                res.append(rdma)
            return res

        def wait_half(h):
            for off in range(1, N_DEV):
                src = (my_pos + off) % N_DEV
                recv = pltpu.make_async_remote_copy(
                    src_ref=gather_ref.at[h, src],
                    dst_ref=gather_ref.at[h, src],
                    send_sem=send_sems.at[h * N_DEV],
                    recv_sem=recv_sems.at[h * N_DEV + src],
                    device_id=(src,),
                    device_id_type=pl.DeviceIdType.MESH,
                )
                recv.wait_recv()

        sends = []
        for h in range(H):
            sl = pl.ds(h * rows_h, rows_h)
            ev = jnp.exp(x_ref[sl, :])
            s_col = jnp.sum(ev, axis=1, keepdims=True)
            e_ref[sl, :] = ev.astype(jnp.bfloat16)
            gather_ref[h, my_pos] = jnp.swapaxes(s_col, 0, 1).reshape(
                stat_sub, STAT_LANE
            )
            if h == 0:
                pl.semaphore_wait(barrier_sem, N_DEV - 1)
            sends += send_half(h)

        for h in range(H):
            wait_half(h)
            s16 = (gather_ref[h, 0] + gather_ref[h, 1]
                   + gather_ref[h, 2] + gather_ref[h, 3])
            inv_col = jnp.swapaxes((1.0 / s16).reshape(1, rows_h), 0, 1)
            sl = pl.ds(h * rows_h, rows_h)
            out_ref[sl, :] = (
                e_ref[sl, :].astype(jnp.float32) * inv_col
            ).astype(jnp.bfloat16)

        for rdma in sends:
            rdma.wait_send()

    return pl.pallas_call(
        body,
        out_shape=jax.ShapeDtypeStruct((m_rows, n_cols), jnp.bfloat16),
        in_specs=[pl.BlockSpec(memory_space=pltpu.VMEM)],
        out_specs=pl.BlockSpec(memory_space=pltpu.VMEM),
        scratch_shapes=[
            pltpu.VMEM((m_rows, n_cols), jnp.bfloat16),
            pltpu.VMEM((H, N_DEV, m_rows // H // STAT_LANE, STAT_LANE),
                       jnp.float32),
            pltpu.SemaphoreType.DMA((H * N_DEV,)),
            pltpu.SemaphoreType.DMA((H * N_DEV,)),
        ],
        compiler_params=pltpu.CompilerParams(
            collective_id=0, vmem_limit_bytes=64 * 1024 * 1024
        ),
    )(x)
